# baseline (speedup 1.0000x reference)
import functools

import jax
import jax.numpy as jnp
from jax import lax
from jax.experimental import pallas as pl
from jax.experimental.pallas import tpu as pltpu
from jax.experimental.pallas import tpu_sc as plsc

F32 = jnp.float32
I32 = jnp.int32

_NC = 2
_NS = 16
_CE = 128
_CW = 64
_NBUF = 4


@functools.lru_cache(maxsize=None)
def _make_agg(n, nch, te_ch):
    assert nch % _NC == 0 and n % 8 == 0
    cpc = nch // _NC
    wrows = (n // _NS) & ~7
    wlast = n - (_NS - 1) * wrows
    zrows = -(-(n + 1) // (8 * _NS)) * 8
    acc_rows = zrows * _NS

    mesh = plsc.VectorSubcoreMesh(core_axis_name="c", subcore_axis_name="s")

    assert te_ch % _NBUF == 0
    out_type = [jax.ShapeDtypeStruct((nch, n, _CW), F32)]
    scratch = [
        pltpu.VMEM((te_ch, _CE), I32),
        pltpu.VMEM((te_ch, _CE), I32),
        [pltpu.VMEM((_CE, _CW), F32) for _ in range(_NBUF)],
        [pltpu.SemaphoreType.DMA for _ in range(_NBUF)],
        [pltpu.SemaphoreType.DMA for _ in range(_NBUF)],
        pltpu.VMEM((_CE, _CW), F32),
        pltpu.VMEM_SHARED((acc_rows, _CW), F32),
    ]

    def body(*refs):
        (x_hbm, src_hbm, dst_hbm, out_hbm,
         src_v, dst_v, gbufs, sems, ssems, zbuf, acc) = refs
        cid = lax.axis_index("c")
        sid = lax.axis_index("s")

        pltpu.sync_copy(src_hbm.at[sid], src_v)
        pltpu.sync_copy(dst_hbm.at[sid], dst_v)

        zero16 = jnp.zeros((16,), F32)

        @pl.loop(0, _CE)
        def _(i):
            for j in range(_CW // 16):
                zbuf[i, pl.ds(j * 16, 16)] = zero16

        def zero_slice():
            for r in range(0, zrows, _CE):
                sz = min(_CE, zrows - r)
                off = pl.multiple_of(sid * zrows + r, 8)
                pltpu.sync_copy(zbuf.at[pl.ds(0, sz)],
                                acc.at[pl.ds(off, sz)])

        def write_out(dst_hbm_ref):
            @pl.when(sid < _NS - 1)
            def _():
                off = pl.multiple_of(sid * wrows, 8)
                pltpu.sync_copy(acc.at[pl.ds(off, wrows)],
                                dst_hbm_ref.at[pl.ds(off, wrows)])

            @pl.when(sid == _NS - 1)
            def _():
                off = (_NS - 1) * wrows
                pltpu.sync_copy(acc.at[pl.ds(off, wlast)],
                                dst_hbm_ref.at[pl.ds(off, wlast)])

        for k in range(cpc):
            cc = cid * cpc + k

            def gather_start(j, b):
                pltpu.async_copy(x_hbm.at[cc].at[src_v.at[j]],
                                 gbufs[b], sems[b])

            def gather_wait(j, b):
                pltpu.make_async_copy(x_hbm.at[cc].at[src_v.at[j]],
                                      gbufs[b], sems[b]).wait()

            def scatter_start(j, b):
                pltpu.async_copy(gbufs[b], acc.at[dst_v.at[j]],
                                 ssems[b], add=True)

            def scatter_wait(j, b):
                pltpu.make_async_copy(gbufs[b], acc.at[dst_v.at[j]],
                                      ssems[b]).wait()

            for b in range(_NBUF):
                gather_start(b, b)
            zero_slice()
            plsc.subcore_barrier()

            @pl.loop(0, te_ch, step=_NBUF)
            def _(j):
                for b in range(_NBUF):
                    gather_wait(j + b, b)
                    scatter_start(j + b, b)
                for b in range(_NBUF):
                    @pl.when(j + b + _NBUF < te_ch)
                    def _():
                        scatter_wait(j + b, b)
                        gather_start(j + b + _NBUF, b)

            for b in range(_NBUF):
                scatter_wait(te_ch - _NBUF + b, b)
            plsc.subcore_barrier()
            write_out(out_hbm.at[cc])
            plsc.subcore_barrier()

    return pl.kernel(body, out_type=out_type, mesh=mesh,
                     scratch_types=scratch,
                     compiler_params=pltpu.CompilerParams(
                         use_tc_tiling_on_sc=False))


@functools.lru_cache(maxsize=None)
def _make_counts(n, te_ch):
    assert n % 8 == 0
    wrows = (n // _NS) & ~7
    wlast = n - (_NS - 1) * wrows
    zrows = -(-(n + 1) // (8 * _NS)) * 8
    acc_rows = zrows * _NS

    mesh = plsc.VectorSubcoreMesh(core_axis_name="c", subcore_axis_name="s")
    scratch = [
        pltpu.VMEM((te_ch, _CE), I32),
        pltpu.VMEM((_CE, 16), F32),
        pltpu.VMEM_SHARED((acc_rows, 16), F32),
    ]

    def body(dst_hbm, cnt_hbm, dst_v, ones_v, cacc):
        cid = lax.axis_index("c")
        sid = lax.axis_index("s")

        @pl.when(cid == 0)
        def _():
            pltpu.sync_copy(dst_hbm.at[sid], dst_v)
            zero16 = jnp.zeros((16,), F32)
            one16 = jnp.ones((16,), F32)

            @pl.loop(0, _CE)
            def _(i):
                ones_v[i, :] = zero16

            for r in range(0, zrows, _CE):
                sz = min(_CE, zrows - r)
                off = pl.multiple_of(sid * zrows + r, 8)
                pltpu.sync_copy(ones_v.at[pl.ds(0, sz)],
                                cacc.at[pl.ds(off, sz)])

            @pl.loop(0, _CE)
            def _(i):
                ones_v[i, :] = one16

            plsc.subcore_barrier()

            @pl.loop(0, te_ch)
            def _(j):
                pltpu.sync_copy(ones_v, cacc.at[dst_v.at[j]], add=True)

            plsc.subcore_barrier()

            @pl.when(sid < _NS - 1)
            def _():
                off = pl.multiple_of(sid * wrows, 8)
                pltpu.sync_copy(cacc.at[pl.ds(off, wrows)],
                                cnt_hbm.at[pl.ds(off, wrows)])

            @pl.when(sid == _NS - 1)
            def _():
                off = (_NS - 1) * wrows
                pltpu.sync_copy(cacc.at[pl.ds(off, wlast)],
                                cnt_hbm.at[pl.ds(off, wlast)])

    return pl.kernel(body, out_type=jax.ShapeDtypeStruct((n, 16), F32),
                     mesh=mesh, scratch_types=scratch,
                     compiler_params=pltpu.CompilerParams(
                         use_tc_tiling_on_sc=False))


def _pick_rows(n):
    for d in range(512, 7, -8):
        if n % d == 0:
            return d
    return n


def _concat_chunks(ref):
    return jnp.concatenate([ref[c] for c in range(ref.shape[0])], axis=1)


def _inv_deg(cnt_ref):
    return 1.0 / jnp.maximum(cnt_ref[:, 0:1], 1.0)


def _chunk_spec(nch, rows):
    return pl.BlockSpec((nch, rows, _CW), lambda i: (0, i, 0))


def _write_chunks(out_ref, h):
    for c in range(out_ref.shape[0]):
        out_ref[c] = h[:, c * _CW:(c + 1) * _CW]


def _sage_body(agg_ref, cnt_ref, x_ref, wl_ref, bl_ref, wr_ref,
               hn_ref, hc_ref):
    mean = _concat_chunks(agg_ref) * _inv_deg(cnt_ref)
    h = (jnp.dot(mean, wl_ref[...], preferred_element_type=F32) + bl_ref[...]
         + jnp.dot(x_ref[...], wr_ref[...], preferred_element_type=F32))
    h = jnp.maximum(h, 0.0)
    hn_ref[...] = h
    _write_chunks(hc_ref, h)


def _sage_tail_body(agg_ref, cnt_ref, x_ref, wl1_ref, bl1_ref, wr1_ref,
                    wl2_ref, bl2_ref, wr2_ref, p_ref, q_ref):
    mean = _concat_chunks(agg_ref) * _inv_deg(cnt_ref)
    h1 = (jnp.dot(mean, wl1_ref[...], preferred_element_type=F32)
          + bl1_ref[...]
          + jnp.dot(x_ref[...], wr1_ref[...], preferred_element_type=F32))
    h1 = jnp.maximum(h1, 0.0)
    p = jnp.dot(h1, wl2_ref[...], preferred_element_type=F32)
    q_ref[...] = (jnp.dot(h1, wr2_ref[...], preferred_element_type=F32)
                  + bl2_ref[...])
    _write_chunks(p_ref, p)


def _final_body(agg_ref, cnt_ref, q_ref, h_ref, s_ref):
    out = _concat_chunks(agg_ref) * _inv_deg(cnt_ref) + q_ref[...]
    h_ref[...] = out
    s_ref[...] = 1.0 / (1.0 + jnp.exp(-out))


@functools.lru_cache(maxsize=None)
def _make_tc_layer(n, din, dout):
    rows = _pick_rows(n)
    nci, nco = din // _CW, dout // _CW
    return pl.pallas_call(
        _sage_body,
        grid=(n // rows,),
        in_specs=[
            _chunk_spec(nci, rows),
            pl.BlockSpec((rows, 16), lambda i: (i, 0)),
            pl.BlockSpec((rows, din), lambda i: (i, 0)),
            pl.BlockSpec((din, dout), lambda i: (0, 0)),
            pl.BlockSpec((1, dout), lambda i: (0, 0)),
            pl.BlockSpec((din, dout), lambda i: (0, 0)),
        ],
        out_specs=[
            pl.BlockSpec((rows, dout), lambda i: (i, 0)),
            _chunk_spec(nco, rows),
        ],
        out_shape=[
            jax.ShapeDtypeStruct((n, dout), F32),
            jax.ShapeDtypeStruct((nco, n, _CW), F32),
        ],
    )


@functools.lru_cache(maxsize=None)
def _make_tc_tail(n, din, dmid, dout):
    rows = _pick_rows(n)
    nci, nco = din // _CW, dout // _CW
    return pl.pallas_call(
        _sage_tail_body,
        grid=(n // rows,),
        in_specs=[
            _chunk_spec(nci, rows),
            pl.BlockSpec((rows, 16), lambda i: (i, 0)),
            pl.BlockSpec((rows, din), lambda i: (i, 0)),
            pl.BlockSpec((din, dmid), lambda i: (0, 0)),
            pl.BlockSpec((1, dmid), lambda i: (0, 0)),
            pl.BlockSpec((din, dmid), lambda i: (0, 0)),
            pl.BlockSpec((dmid, dout), lambda i: (0, 0)),
            pl.BlockSpec((1, dout), lambda i: (0, 0)),
            pl.BlockSpec((dmid, dout), lambda i: (0, 0)),
        ],
        out_specs=[
            _chunk_spec(nco, rows),
            pl.BlockSpec((rows, dout), lambda i: (i, 0)),
        ],
        out_shape=[
            jax.ShapeDtypeStruct((nco, n, _CW), F32),
            jax.ShapeDtypeStruct((n, dout), F32),
        ],
    )


@functools.lru_cache(maxsize=None)
def _make_tc_final(n, dout):
    rows = _pick_rows(n)
    nci = dout // _CW
    return pl.pallas_call(
        _final_body,
        grid=(n // rows,),
        in_specs=[
            _chunk_spec(nci, rows),
            pl.BlockSpec((rows, 16), lambda i: (i, 0)),
            pl.BlockSpec((rows, dout), lambda i: (i, 0)),
        ],
        out_specs=[
            pl.BlockSpec((rows, dout), lambda i: (i, 0)),
            pl.BlockSpec((rows, dout), lambda i: (i, 0)),
        ],
        out_shape=[
            jax.ShapeDtypeStruct((n, dout), F32),
            jax.ShapeDtypeStruct((n, dout), F32),
        ],
    )


def _chunked(a):
    n, d = a.shape
    return a.reshape(n, d // _CW, _CW).transpose(1, 0, 2)


def kernel(x, edge_index, Wl0, bl0, Wr0, Wl1, bl1, Wr1, Wl2, bl2, Wr2):
    n, din = x.shape
    e = edge_index.shape[1]
    dh = Wl0.shape[0]
    dout = Wl2.shape[0]

    te_ch = -(-(-(-e // (_NS * _CE))) // _NBUF) * _NBUF
    pe = _NS * te_ch * _CE
    src = jnp.concatenate(
        [edge_index[0], jnp.zeros((pe - e,), I32)]).reshape(_NS, te_ch, _CE)
    dst = jnp.concatenate(
        [edge_index[1], jnp.full((pe - e,), n, I32)]).reshape(_NS, te_ch, _CE)

    cnt = _make_counts(n, te_ch)(dst)
    agg0, = _make_agg(n, din // _CW, te_ch)(_chunked(x), src, dst)
    h0, h0c = _make_tc_layer(n, din, dh)(
        agg0, cnt, x, Wl0.T, bl0[None], Wr0.T)
    agg1, = _make_agg(n, dh // _CW, te_ch)(h0c, src, dst)
    pc, q = _make_tc_tail(n, dh, dh, dout)(
        agg1, cnt, h0, Wl1.T, bl1[None], Wr1.T, Wl2.T, bl2[None], Wr2.T)
    agg2, = _make_agg(n, dout // _CW, te_ch)(pc, src, dst)
    h, s = _make_tc_final(n, dout)(agg2, cnt, q)
    return (h, s)


# speedup vs baseline: 1.1290x; 1.1290x over previous
import functools

import jax
import jax.numpy as jnp
from jax import lax
from jax.experimental import pallas as pl
from jax.experimental.pallas import tpu as pltpu
from jax.experimental.pallas import tpu_sc as plsc

F32 = jnp.float32
I32 = jnp.int32

_NC = 2
_NS = 16
_CE = 128
_CW = 128
_NBUF = 2
_NIDX = 4


@functools.lru_cache(maxsize=None)
def _make_agg(n, nch, te_ch):
    assert nch % _NC == 0 and n % 8 == 0
    cpc = nch // _NC
    wrows = (n // _NS) & ~7
    wlast = n - (_NS - 1) * wrows
    zrows = -(-(n + 1) // (8 * _NS)) * 8
    acc_rows = zrows * _NS

    mesh = plsc.VectorSubcoreMesh(core_axis_name="c", subcore_axis_name="s")

    assert te_ch % _NIDX == 0
    zb = 64
    out_type = [jax.ShapeDtypeStruct((nch, n, _CW), F32)]
    scratch = [
        pltpu.VMEM((_NIDX, _CE), I32),
        pltpu.VMEM((_NIDX, _CE), I32),
        [pltpu.VMEM((_CE, _CW), F32) for _ in range(_NBUF)],
        [pltpu.SemaphoreType.DMA for _ in range(_NBUF)],
        [pltpu.SemaphoreType.DMA for _ in range(_NIDX)],
        pltpu.VMEM((zb, _CW), F32),
        pltpu.VMEM_SHARED((acc_rows, _CW), F32),
    ]

    def body(*refs):
        (x_hbm, src_hbm, dst_hbm, out_hbm,
         sidx, didx, gbufs, gsems, isems, zbuf, acc) = refs
        cid = lax.axis_index("c")
        sid = lax.axis_index("s")

        def idx_start(t, s):
            pltpu.async_copy(src_hbm.at[sid].at[t], sidx.at[s], isems[s])
            pltpu.async_copy(dst_hbm.at[sid].at[t], didx.at[s], isems[s])

        def idx_wait(t, s):
            pltpu.make_async_copy(src_hbm.at[sid].at[t], sidx.at[s],
                                  isems[s]).wait()
            pltpu.make_async_copy(dst_hbm.at[sid].at[t], didx.at[s],
                                  isems[s]).wait()

        zero16 = jnp.zeros((16,), F32)

        @pl.loop(0, zb)
        def _(i):
            for j in range(_CW // 16):
                zbuf[i, pl.ds(j * 16, 16)] = zero16

        def zero_slice():
            for r in range(0, zrows, zb):
                sz = min(zb, zrows - r)
                off = pl.multiple_of(sid * zrows + r, 8)
                pltpu.sync_copy(zbuf.at[pl.ds(0, sz)],
                                acc.at[pl.ds(off, sz)])

        def write_out(dst_hbm_ref):
            @pl.when(sid < _NS - 1)
            def _():
                off = pl.multiple_of(sid * wrows, 8)
                pltpu.sync_copy(acc.at[pl.ds(off, wrows)],
                                dst_hbm_ref.at[pl.ds(off, wrows)])

            @pl.when(sid == _NS - 1)
            def _():
                off = (_NS - 1) * wrows
                pltpu.sync_copy(acc.at[pl.ds(off, wlast)],
                                dst_hbm_ref.at[pl.ds(off, wlast)])

        for k in range(cpc):
            cc = cid * cpc + k

            def gather_start(j, s, b):
                pltpu.async_copy(x_hbm.at[cc].at[sidx.at[s]],
                                 gbufs[b], gsems[b])

            def gather_wait(j, s, b):
                pltpu.make_async_copy(x_hbm.at[cc].at[sidx.at[s]],
                                      gbufs[b], gsems[b]).wait()

            for s in range(_NIDX):
                idx_start(s, s)
            for b in range(_NBUF):
                idx_wait(b, b)
                gather_start(b, b, b)
            zero_slice()
            plsc.subcore_barrier()

            @pl.loop(0, te_ch, step=_NIDX)
            def _(j):
                for u in range(_NIDX):
                    t = j + u
                    b = u % _NBUF
                    gather_wait(t, u, b)
                    pltpu.sync_copy(gbufs[b], acc.at[didx.at[u]], add=True)

                    @pl.when(t + _NIDX < te_ch)
                    def _():
                        idx_start(t + _NIDX, u)

                    @pl.when(t + _NBUF < te_ch)
                    def _():
                        idx_wait(t + _NBUF, (u + _NBUF) % _NIDX)
                        gather_start(t + _NBUF, (u + _NBUF) % _NIDX, b)

            plsc.subcore_barrier()
            write_out(out_hbm.at[cc])
            plsc.subcore_barrier()

    return pl.kernel(body, out_type=out_type, mesh=mesh,
                     scratch_types=scratch,
                     compiler_params=pltpu.CompilerParams(
                         use_tc_tiling_on_sc=False))


@functools.lru_cache(maxsize=None)
def _make_counts(n, te_ch):
    assert n % 8 == 0
    wrows = (n // _NS) & ~7
    wlast = n - (_NS - 1) * wrows
    zrows = -(-(n + 1) // (8 * _NS)) * 8
    acc_rows = zrows * _NS

    mesh = plsc.VectorSubcoreMesh(core_axis_name="c", subcore_axis_name="s")
    scratch = [
        pltpu.VMEM((te_ch, _CE), I32),
        pltpu.VMEM((_CE, 16), F32),
        pltpu.VMEM_SHARED((acc_rows, 16), F32),
    ]

    def body(dst_hbm, cnt_hbm, dst_v, ones_v, cacc):
        cid = lax.axis_index("c")
        sid = lax.axis_index("s")

        @pl.when(cid == 0)
        def _():
            pltpu.sync_copy(dst_hbm.at[sid], dst_v)
            zero16 = jnp.zeros((16,), F32)
            one16 = jnp.ones((16,), F32)

            @pl.loop(0, _CE)
            def _(i):
                ones_v[i, :] = zero16

            for r in range(0, zrows, _CE):
                sz = min(_CE, zrows - r)
                off = pl.multiple_of(sid * zrows + r, 8)
                pltpu.sync_copy(ones_v.at[pl.ds(0, sz)],
                                cacc.at[pl.ds(off, sz)])

            @pl.loop(0, _CE)
            def _(i):
                ones_v[i, :] = one16

            plsc.subcore_barrier()

            @pl.loop(0, te_ch)
            def _(j):
                pltpu.sync_copy(ones_v, cacc.at[dst_v.at[j]], add=True)

            plsc.subcore_barrier()

            @pl.when(sid < _NS - 1)
            def _():
                off = pl.multiple_of(sid * wrows, 8)
                pltpu.sync_copy(cacc.at[pl.ds(off, wrows)],
                                cnt_hbm.at[pl.ds(off, wrows)])

            @pl.when(sid == _NS - 1)
            def _():
                off = (_NS - 1) * wrows
                pltpu.sync_copy(cacc.at[pl.ds(off, wlast)],
                                cnt_hbm.at[pl.ds(off, wlast)])

    return pl.kernel(body, out_type=jax.ShapeDtypeStruct((n, 16), F32),
                     mesh=mesh, scratch_types=scratch,
                     compiler_params=pltpu.CompilerParams(
                         use_tc_tiling_on_sc=False))


def _pick_rows(n):
    for d in range(512, 7, -8):
        if n % d == 0:
            return d
    return n


def _concat_chunks(ref):
    return jnp.concatenate([ref[c] for c in range(ref.shape[0])], axis=1)


def _inv_deg(cnt_ref):
    return 1.0 / jnp.maximum(cnt_ref[:, 0:1], 1.0)


def _chunk_spec(nch, rows):
    return pl.BlockSpec((nch, rows, _CW), lambda i: (0, i, 0))


def _write_chunks(out_ref, h):
    for c in range(out_ref.shape[0]):
        out_ref[c] = h[:, c * _CW:(c + 1) * _CW]


def _sage_body(agg_ref, cnt_ref, x_ref, wl_ref, bl_ref, wr_ref,
               hn_ref, hc_ref):
    mean = _concat_chunks(agg_ref) * _inv_deg(cnt_ref)
    h = (jnp.dot(mean, wl_ref[...], preferred_element_type=F32) + bl_ref[...]
         + jnp.dot(x_ref[...], wr_ref[...], preferred_element_type=F32))
    h = jnp.maximum(h, 0.0)
    hn_ref[...] = h
    _write_chunks(hc_ref, h)


def _sage_tail_body(agg_ref, cnt_ref, x_ref, wl1_ref, bl1_ref, wr1_ref,
                    wl2_ref, bl2_ref, wr2_ref, p_ref, q_ref):
    mean = _concat_chunks(agg_ref) * _inv_deg(cnt_ref)
    h1 = (jnp.dot(mean, wl1_ref[...], preferred_element_type=F32)
          + bl1_ref[...]
          + jnp.dot(x_ref[...], wr1_ref[...], preferred_element_type=F32))
    h1 = jnp.maximum(h1, 0.0)
    p = jnp.dot(h1, wl2_ref[...], preferred_element_type=F32)
    q_ref[...] = (jnp.dot(h1, wr2_ref[...], preferred_element_type=F32)
                  + bl2_ref[...])
    _write_chunks(p_ref, p)


def _final_body(agg_ref, cnt_ref, q_ref, h_ref, s_ref):
    out = _concat_chunks(agg_ref) * _inv_deg(cnt_ref) + q_ref[...]
    h_ref[...] = out
    s_ref[...] = 1.0 / (1.0 + jnp.exp(-out))


@functools.lru_cache(maxsize=None)
def _make_tc_layer(n, din, dout):
    rows = _pick_rows(n)
    nci, nco = din // _CW, dout // _CW
    return pl.pallas_call(
        _sage_body,
        grid=(n // rows,),
        in_specs=[
            _chunk_spec(nci, rows),
            pl.BlockSpec((rows, 16), lambda i: (i, 0)),
            pl.BlockSpec((rows, din), lambda i: (i, 0)),
            pl.BlockSpec((din, dout), lambda i: (0, 0)),
            pl.BlockSpec((1, dout), lambda i: (0, 0)),
            pl.BlockSpec((din, dout), lambda i: (0, 0)),
        ],
        out_specs=[
            pl.BlockSpec((rows, dout), lambda i: (i, 0)),
            _chunk_spec(nco, rows),
        ],
        out_shape=[
            jax.ShapeDtypeStruct((n, dout), F32),
            jax.ShapeDtypeStruct((nco, n, _CW), F32),
        ],
    )


@functools.lru_cache(maxsize=None)
def _make_tc_tail(n, din, dmid, dout):
    rows = _pick_rows(n)
    nci, nco = din // _CW, dout // _CW
    return pl.pallas_call(
        _sage_tail_body,
        grid=(n // rows,),
        in_specs=[
            _chunk_spec(nci, rows),
            pl.BlockSpec((rows, 16), lambda i: (i, 0)),
            pl.BlockSpec((rows, din), lambda i: (i, 0)),
            pl.BlockSpec((din, dmid), lambda i: (0, 0)),
            pl.BlockSpec((1, dmid), lambda i: (0, 0)),
            pl.BlockSpec((din, dmid), lambda i: (0, 0)),
            pl.BlockSpec((dmid, dout), lambda i: (0, 0)),
            pl.BlockSpec((1, dout), lambda i: (0, 0)),
            pl.BlockSpec((dmid, dout), lambda i: (0, 0)),
        ],
        out_specs=[
            _chunk_spec(nco, rows),
            pl.BlockSpec((rows, dout), lambda i: (i, 0)),
        ],
        out_shape=[
            jax.ShapeDtypeStruct((nco, n, _CW), F32),
            jax.ShapeDtypeStruct((n, dout), F32),
        ],
    )


@functools.lru_cache(maxsize=None)
def _make_tc_final(n, dout):
    rows = _pick_rows(n)
    nci = dout // _CW
    return pl.pallas_call(
        _final_body,
        grid=(n // rows,),
        in_specs=[
            _chunk_spec(nci, rows),
            pl.BlockSpec((rows, 16), lambda i: (i, 0)),
            pl.BlockSpec((rows, dout), lambda i: (i, 0)),
        ],
        out_specs=[
            pl.BlockSpec((rows, dout), lambda i: (i, 0)),
            pl.BlockSpec((rows, dout), lambda i: (i, 0)),
        ],
        out_shape=[
            jax.ShapeDtypeStruct((n, dout), F32),
            jax.ShapeDtypeStruct((n, dout), F32),
        ],
    )


def _chunked(a):
    n, d = a.shape
    return a.reshape(n, d // _CW, _CW).transpose(1, 0, 2)


def kernel(x, edge_index, Wl0, bl0, Wr0, Wl1, bl1, Wr1, Wl2, bl2, Wr2):
    n, din = x.shape
    e = edge_index.shape[1]
    dh = Wl0.shape[0]
    dout = Wl2.shape[0]

    te_ch = -(-(-(-e // (_NS * _CE))) // _NBUF) * _NBUF
    pe = _NS * te_ch * _CE
    src = jnp.concatenate(
        [edge_index[0], jnp.zeros((pe - e,), I32)]).reshape(_NS, te_ch, _CE)
    dst = jnp.concatenate(
        [edge_index[1], jnp.full((pe - e,), n, I32)]).reshape(_NS, te_ch, _CE)

    cnt = _make_counts(n, te_ch)(dst)
    agg0, = _make_agg(n, din // _CW, te_ch)(_chunked(x), src, dst)
    h0, h0c = _make_tc_layer(n, din, dh)(
        agg0, cnt, x, Wl0.T, bl0[None], Wr0.T)
    agg1, = _make_agg(n, dh // _CW, te_ch)(h0c, src, dst)
    pc, q = _make_tc_tail(n, dh, dh, dout)(
        agg1, cnt, h0, Wl1.T, bl1[None], Wr1.T, Wl2.T, bl2[None], Wr2.T)
    agg2, = _make_agg(n, dout // _CW, te_ch)(pc, src, dst)
    h, s = _make_tc_final(n, dout)(agg2, cnt, q)
    return (h, s)
